# baseline (speedup 1.0000x reference)
import functools
import jax
import jax.numpy as jnp
from jax import lax
from jax.experimental import pallas as pl
from jax.experimental.pallas import tpu as pltpu
from jax.experimental.pallas import tpu_sc as plsc

N = 128
OUTW = 3 * N
Z0 = 2.0
DZ = 4.0 / 127.0
NC, NS, L = 2, 16, 16
NW = NC * NS
CH = 64


def _sc_search(d_hbm, u_hbm, z_hbm, d_v, u_v, z_v, c_v):
    wid = lax.axis_index("s") * NC + lax.axis_index("c")
    rays = d_hbm.shape[0]
    rpw = rays // NW
    base = wid * rpw

    def chunk(i, _):
        row0 = base + i * CH
        pltpu.sync_copy(d_hbm.at[pl.ds(row0, CH)], d_v)
        pltpu.sync_copy(u_hbm.at[pl.ds(row0, CH)], u_v)

        def ray(r, _):
            carry = 0.0
            for v in range(8):
                dv = d_v[r, pl.ds(16 * v, 16)] + 1e-5
                cs = plsc.cumsum(dv) + carry
                c_v[r, pl.ds(16 * v, 16)] = cs
                carry = cs[15]
            s_tot = carry
            r_vec = jnp.full((16,), 0, dtype=jnp.int32) + r
            for kv in range(8):
                t = u_v[r, pl.ds(16 * kv, 16)] * s_tot
                pos = jnp.zeros((16,), jnp.int32)
                for s in (64, 32, 16, 8, 4, 2, 1):
                    probe = pos + (s - 1)
                    val = plsc.load_gather(c_v, [r_vec, probe])
                    pos = pos + jnp.where(val <= t, s, 0)
                z_v[r, pl.ds(16 * kv, 16)] = Z0 + pos.astype(jnp.float32) * DZ
            return 0

        lax.fori_loop(0, CH, ray, 0)
        pltpu.sync_copy(z_v, z_hbm.at[pl.ds(row0, CH)])
        return 0

    lax.fori_loop(0, rpw // CH, chunk, 0)


def _tc_expand(z_ref, o_ref, dir_ref, pts_ref):
    z = z_ref[...]
    B = z.shape[0]
    kk = lax.broadcasted_iota(jnp.int32, (N, OUTW), 0)
    ll = lax.broadcasted_iota(jnp.int32, (N, OUTW), 1)
    rep = (ll // 3 == kk).astype(jnp.float32)
    z384 = lax.dot_general(
        z, rep, (((1,), (0,)), ((), ())),
        preferred_element_type=jnp.float32,
        precision=lax.Precision.HIGHEST,
    )
    c_i = lax.broadcasted_iota(jnp.int32, (B, OUTW), 1) % 3
    o384 = jnp.where(c_i == 0, o_ref[:, 0:1],
                     jnp.where(c_i == 1, o_ref[:, 1:2], o_ref[:, 2:3]))
    d384 = jnp.where(c_i == 0, dir_ref[:, 0:1],
                     jnp.where(c_i == 1, dir_ref[:, 1:2], dir_ref[:, 2:3]))
    pts_ref[...] = o384 + z384 * d384


@jax.jit
def kernel(origins, directions, density, u):
    O = density.shape[0]

    sc_call = pl.kernel(
        _sc_search,
        out_type=jax.ShapeDtypeStruct((O, N), jnp.float32),
        mesh=plsc.VectorSubcoreMesh(core_axis_name="c", subcore_axis_name="s",
                                    num_cores=NC, num_subcores=NS),
        compiler_params=pltpu.CompilerParams(needs_layout_passes=False),
        scratch_types=[
            pltpu.VMEM((CH, N), jnp.float32),
            pltpu.VMEM((CH, N), jnp.float32),
            pltpu.VMEM((CH, N), jnp.float32),
            pltpu.VMEM((CH, N), jnp.float32),
        ],
    )
    z = sc_call(density, u)

    B = 512
    pts_flat = pl.pallas_call(
        _tc_expand,
        grid=(O // B,),
        in_specs=[
            pl.BlockSpec((B, N), lambda i: (i, 0)),
            pl.BlockSpec((B, 3), lambda i: (i, 0)),
            pl.BlockSpec((B, 3), lambda i: (i, 0)),
        ],
        out_specs=pl.BlockSpec((B, OUTW), lambda i: (i, 0)),
        out_shape=jax.ShapeDtypeStruct((O, OUTW), jnp.float32),
    )(z, origins, directions)
    return pts_flat.reshape(O, N, 3), z.reshape(O, N, 1)


# speedup vs baseline: 2.0898x; 2.0898x over previous
import functools
import jax
import jax.numpy as jnp
from jax import lax
from jax.experimental import pallas as pl
from jax.experimental.pallas import tpu as pltpu
from jax.experimental.pallas import tpu_sc as plsc

N = 128
OUTW = 3 * N
Z0 = 2.0
DZ = 4.0 / 127.0
NC, NS, L = 2, 16, 16
NW = NC * NS
CH = 128


def _sc_search(d_hbm, u_hbm, z_hbm, d_v, u_v, z_v, c_v):
    wid = lax.axis_index("s") * NC + lax.axis_index("c")
    rays = d_hbm.shape[0]
    rpw = rays // NW
    base = wid * rpw

    def chunk(i, _):
        row0 = base + i * CH
        pltpu.sync_copy(d_hbm.at[pl.ds(row0, CH)], d_v)
        pltpu.sync_copy(u_hbm.at[pl.ds(row0, CH)], u_v)

        @plsc.parallel_loop(0, CH, unroll=2)
        def cdf_loop(r):
            carry = 0.0
            for v in range(8):
                dv = d_v[r, pl.ds(16 * v, 16)] + 1e-5
                cs = plsc.cumsum(dv) + carry
                c_v[r, pl.ds(16 * v, 16)] = cs
                carry = cs[15]

        @plsc.parallel_loop(0, CH * 8, unroll=4)
        def search_loop(it):
            r = it >> 3
            kv = it & 7
            r_vec = jnp.full((16,), r, dtype=jnp.int32)
            s_tot = plsc.load_gather(c_v, [r_vec, jnp.full((16,), 127, jnp.int32)])
            t = u_v[r, pl.ds(kv * 16, 16)] * s_tot
            pos = jnp.zeros((16,), jnp.int32)
            for s in (64, 32, 16, 8, 4, 2, 1):
                probe = pos + (s - 1)
                val = plsc.load_gather(c_v, [r_vec, probe])
                pos = pos + jnp.where(val <= t, s, 0)
            z_v[r, pl.ds(kv * 16, 16)] = Z0 + pos.astype(jnp.float32) * DZ

        pltpu.sync_copy(z_v, z_hbm.at[pl.ds(row0, CH)])
        return 0

    lax.fori_loop(0, rpw // CH, chunk, 0)


def _tc_expand(z_ref, o_ref, dir_ref, pts_ref):
    z = z_ref[...]
    B = z.shape[0]
    kk = lax.broadcasted_iota(jnp.int32, (N, OUTW), 0)
    ll = lax.broadcasted_iota(jnp.int32, (N, OUTW), 1)
    rep = (ll // 3 == kk).astype(jnp.float32)
    z384 = lax.dot_general(
        z, rep, (((1,), (0,)), ((), ())),
        preferred_element_type=jnp.float32,
        precision=lax.Precision.HIGHEST,
    )
    c_i = lax.broadcasted_iota(jnp.int32, (B, OUTW), 1) % 3
    o384 = jnp.where(c_i == 0, o_ref[:, 0:1],
                     jnp.where(c_i == 1, o_ref[:, 1:2], o_ref[:, 2:3]))
    d384 = jnp.where(c_i == 0, dir_ref[:, 0:1],
                     jnp.where(c_i == 1, dir_ref[:, 1:2], dir_ref[:, 2:3]))
    pts_ref[...] = o384 + z384 * d384


@jax.jit
def kernel(origins, directions, density, u):
    O = density.shape[0]

    sc_call = pl.kernel(
        _sc_search,
        out_type=jax.ShapeDtypeStruct((O, N), jnp.float32),
        mesh=plsc.VectorSubcoreMesh(core_axis_name="c", subcore_axis_name="s",
                                    num_cores=NC, num_subcores=NS),
        compiler_params=pltpu.CompilerParams(needs_layout_passes=False),
        scratch_types=[
            pltpu.VMEM((CH, N), jnp.float32),
            pltpu.VMEM((CH, N), jnp.float32),
            pltpu.VMEM((CH, N), jnp.float32),
            pltpu.VMEM((CH, N), jnp.float32),
        ],
    )
    z = sc_call(density, u)

    B = 512
    pts_flat = pl.pallas_call(
        _tc_expand,
        grid=(O // B,),
        in_specs=[
            pl.BlockSpec((B, N), lambda i: (i, 0)),
            pl.BlockSpec((B, 3), lambda i: (i, 0)),
            pl.BlockSpec((B, 3), lambda i: (i, 0)),
        ],
        out_specs=pl.BlockSpec((B, OUTW), lambda i: (i, 0)),
        out_shape=jax.ShapeDtypeStruct((O, OUTW), jnp.float32),
    )(z, origins, directions)
    return pts_flat.reshape(O, N, 3), z.reshape(O, N, 1)
